# Initial kernel scaffold; baseline (speedup 1.0000x reference)
import jax
import jax.numpy as jnp
from jax.experimental import pallas as pl


def kernel(x, fc_weight, bias):
    raise NotImplementedError("write your pallas kernel here")


# speedup vs baseline: 1.4820x; 1.4820x over previous
import functools

import jax
import jax.numpy as jnp
from jax import lax
from jax.experimental import pallas as pl
from jax.experimental.pallas import tpu as pltpu
from jax.experimental.pallas import tpu_sc as plsc

B = 16384
F = 26
TABLE = 40000
L = 16

_info = plsc.get_sparse_core_info()
NC, NS = _info.num_cores, _info.num_subcores
NW = NC * NS
BPW = B // NW
CHUNK = BPW * F
NV = BPW // L


def _sc_body(xt_hbm, w_hbm, bias_hbm, out_hbm, idxv, valv, outv, bv, sem):
    wid = lax.axis_index("s") * NC + lax.axis_index("c")
    base = wid * BPW

    copies = [
        pltpu.async_copy(
            xt_hbm.at[pl.ds(f * B + base, BPW)],
            idxv.at[pl.ds(f * BPW, BPW)],
            sem,
        )
        for f in range(F)
    ]
    pltpu.sync_copy(bias_hbm, bv)
    for c in copies:
        c.wait()

    def build(j, _):
        for f in range(1, F):
            s = pl.ds(f * BPW + j * L, L)
            idxv[s] = idxv[s] + (f * TABLE)
        return 0

    lax.fori_loop(0, NV, build, 0)

    pltpu.async_copy(w_hbm.at[idxv], valv, sem).wait()

    bias_vec = bv[...]

    def reduce(j, _):
        acc = bias_vec
        for f in range(F):
            acc = acc + valv[pl.ds(f * BPW + j * L, L)]
        outv[pl.ds(j * L, L)] = acc
        return 0

    lax.fori_loop(0, NV, reduce, 0)

    pltpu.sync_copy(outv, out_hbm.at[pl.ds(base, BPW)])


@functools.partial(jax.jit, static_argnames=())
def kernel(x, fc_weight, bias):
    xt1 = x.astype(jnp.int32).T.reshape(F * B)
    w1 = fc_weight.reshape(-1)
    bias16 = jnp.broadcast_to(bias.astype(jnp.float32), (L,))

    mesh = plsc.VectorSubcoreMesh(core_axis_name="c", subcore_axis_name="s")
    run = pl.kernel(
        _sc_body,
        mesh=mesh,
        out_type=jax.ShapeDtypeStruct((B,), jnp.float32),
        scratch_types=[
            pltpu.VMEM((CHUNK,), jnp.int32),
            pltpu.VMEM((CHUNK,), jnp.float32),
            pltpu.VMEM((BPW,), jnp.float32),
            pltpu.VMEM((L,), jnp.float32),
            pltpu.SemaphoreType.DMA,
        ],
    )
    out1 = run(xt1, w1, bias16)
    return out1.reshape(B, 1)
